# Initial kernel scaffold; baseline (speedup 1.0000x reference)
import jax
import jax.numpy as jnp
from jax.experimental import pallas as pl


def kernel(loc_data, conf_data, loc_five_data, priors, targets):
    raise NotImplementedError("write your pallas kernel here")


# speedup vs baseline: 8.1043x; 8.1043x over previous
import functools

import jax
import jax.numpy as jnp
from jax.experimental import pallas as pl
from jax.experimental.pallas import tpu as pltpu

_THRESHOLD = 0.35
_NEGPOS_RATIO = 7
_VAR0 = 0.1
_VAR1 = 0.2


def _match_kernel(pt_ref, tg_ref, bto_ref, bti_ref, bpi_ref, sv_ref, si_ref,
                  *, TP, T, O):
    t = pl.program_id(1)

    @pl.when(t == 0)
    def _init():
        sv_ref[...] = jnp.full((O, 1), -1.0, jnp.float32)
        si_ref[...] = jnp.zeros((O, 1), jnp.int32)

    px = pt_ref[0:1, :]
    py = pt_ref[1:2, :]
    pw = pt_ref[2:3, :]
    ph = pt_ref[3:4, :]
    bx1 = px - pw / 2.0
    by1 = py - ph / 2.0
    bx2 = px + pw / 2.0
    by2 = py + ph / 2.0
    area_b = (bx2 - bx1) * (by2 - by1)

    tb = tg_ref[0]
    tx1 = tb[:, 0:1]
    ty1 = tb[:, 1:2]
    tx2 = tb[:, 2:3]
    ty2 = tb[:, 3:4]
    area_a = (tx2 - tx1) * (ty2 - ty1)

    iw = jnp.maximum(jnp.minimum(tx2, bx2) - jnp.maximum(tx1, bx1), 0.0)
    ih = jnp.maximum(jnp.minimum(ty2, by2) - jnp.maximum(ty1, by1), 0.0)
    inter = iw * ih
    iou = inter / (area_a + area_b - inter)

    io_o = jax.lax.broadcasted_iota(jnp.int32, (O, 1), 0)
    io_p = jax.lax.broadcasted_iota(jnp.int32, (1, TP), 1)

    pm = jnp.max(iou, axis=0, keepdims=True)
    pidx = jnp.min(jnp.where(iou == pm, io_o, O), axis=0, keepdims=True)
    bto_ref[0, 0] = pm
    bti_ref[0, 0] = pidx

    tv = jnp.max(iou, axis=1, keepdims=True)
    tl = jnp.min(jnp.where(iou == tv, io_p, TP), axis=1, keepdims=True)
    gidx = t * TP + tl
    sv = sv_ref[...]
    si = si_ref[...]
    upd = tv > sv
    new_sv = jnp.where(upd, tv, sv)
    new_si = jnp.where(upd, gidx, si)
    sv_ref[...] = new_sv
    si_ref[...] = new_si
    bpi_ref[0] = new_si


def _smooth_l1(d):
    ad = jnp.abs(d)
    return jnp.where(ad < 1.0, 0.5 * d * d, ad - 0.5)


def _loss_kernel(loc_ref, conf_ref, five_ref, pr_ref, tg_ref, bto_ref,
                 bti_ref, bpi_ref, rows_ref, part_ref, *, TP, O):
    t = pl.program_id(1)

    bto = bto_ref[0, 0]
    bti = bti_ref[0, 0]
    bpi = bpi_ref[0]

    io_o = jax.lax.broadcasted_iota(jnp.int32, (O, 1), 0)
    gp = t * TP + jax.lax.broadcasted_iota(jnp.int32, (1, TP), 1)

    match = bpi == gp
    forced = jnp.max(match.astype(jnp.int32), axis=0, keepdims=True) > 0
    fidx = jnp.max(jnp.where(match, io_o, -1), axis=0, keepdims=True)
    idx_f = jnp.where(forced, fidx, bti)
    pos_row = jnp.logical_or(forced, bto >= _THRESHOLD)
    posf_row = pos_row.astype(jnp.float32)

    oh = (io_o == idx_f).astype(jnp.float32)
    lhs = jnp.concatenate([oh, posf_row], axis=0)
    tb = tg_ref[0]
    top = jnp.concatenate(
        [tb[:, 0:4], tb[:, 5:15], jnp.zeros((O, 1), jnp.float32)], axis=1)
    bottom = jnp.concatenate(
        [jnp.zeros((1, 14), jnp.float32), jnp.ones((1, 1), jnp.float32)],
        axis=1)
    table = jnp.concatenate([top, bottom], axis=0)
    matched = jax.lax.dot_general(
        lhs, table, (((0,), (0,)), ((), ())),
        preferred_element_type=jnp.float32)

    posf = matched[:, 14:15]
    posb = posf >= 0.5

    pr = pr_ref[...]
    pcxy = pr[:, 0:2]
    pwh = pr[:, 2:4]

    m12 = matched[:, 0:2]
    m34 = matched[:, 2:4]
    g_cxcy = ((m12 + m34) / 2.0 - pcxy) / (_VAR0 * pwh)
    g_wh = jnp.log((m34 - m12) / pwh) / _VAR1
    loc = loc_ref[0]
    sl = _smooth_l1(loc[:, 0:2] - g_cxcy) + _smooth_l1(loc[:, 2:4] - g_wh)
    loss_l = jnp.sum(sl * posf)

    ml = matched[:, 4:14]
    pc_rep = jnp.concatenate([pcxy] * 5, axis=1)
    pwh_rep = jnp.concatenate([pwh] * 5, axis=1)
    g_lm = (ml - pc_rep) / (_VAR0 * pwh_rep)
    dlm = five_ref[0] - g_lm
    loss_coords = jnp.sum(dlm * dlm * posf)

    c = conf_ref[0]
    c0 = c[:, 0:1]
    c1 = c[:, 1:2]
    mx = jnp.maximum(c0, c1)
    lse = mx + jnp.log(jnp.exp(c0 - mx) + jnp.exp(c1 - mx))
    ce = lse - jnp.where(posb, c1, c0)
    pos_ce = jnp.sum(jnp.where(posb, ce, 0.0))
    negloss = jnp.where(posb, 0.0, lse - c0)
    rows_ref[0, 0] = negloss
    npos = jnp.sum(posf)

    lane = jax.lax.broadcasted_iota(jnp.int32, (1, 8), 1)
    vec = (jnp.where(lane == 0, loss_l, 0.0)
           + jnp.where(lane == 1, loss_coords, 0.0)
           + jnp.where(lane == 2, pos_ce, 0.0)
           + jnp.where(lane == 3, npos, 0.0))
    part_ref[0, 0] = vec


def _topk_kernel(rows_ref, part_ref, out_ref, *, P):
    v = rows_ref[0]
    bits = jax.lax.bitcast_convert_type(v, jnp.int32)
    npos = jnp.sum(part_ref[0, :, 0, 3])
    k = jnp.minimum(_NEGPOS_RATIO * npos.astype(jnp.int32), P - 1)

    def body(_, carry):
        lo, hi = carry
        mid = lo + (hi - lo) // 2
        cnt = jnp.sum((bits >= mid).astype(jnp.int32))
        good = cnt >= k
        return (jnp.where(good, mid, lo), jnp.where(good, hi, mid))

    lo, _ = jax.lax.fori_loop(
        0, 31, body, (jnp.int32(0), jnp.int32(0x7F800000)))
    tf = jax.lax.bitcast_convert_type(lo, jnp.float32)
    gt = bits > lo
    cnt_gt = jnp.sum(gt.astype(jnp.float32))
    sum_gt = jnp.sum(jnp.where(gt, v, 0.0))
    topk = sum_gt + (k.astype(jnp.float32) - cnt_gt) * tf
    topk = jnp.where(k >= 1, topk, 0.0)
    lane = jax.lax.broadcasted_iota(jnp.int32, (1, 8), 1)
    out_ref[0] = jnp.where(lane == 0, topk, 0.0)


@jax.jit
def kernel(loc_data, conf_data, loc_five_data, priors, targets):
    B, P, _ = loc_data.shape
    O = targets.shape[1]
    TP = 2048 if P % 2048 == 0 else P
    T = P // TP

    priors_t = priors.T

    bto, bti, bpi = pl.pallas_call(
        functools.partial(_match_kernel, TP=TP, T=T, O=O),
        grid=(B, T),
        in_specs=[
            pl.BlockSpec((4, TP), lambda b, t: (0, t)),
            pl.BlockSpec((1, O, 15), lambda b, t: (b, 0, 0)),
        ],
        out_specs=[
            pl.BlockSpec((1, 1, 1, TP), lambda b, t: (b, t, 0, 0)),
            pl.BlockSpec((1, 1, 1, TP), lambda b, t: (b, t, 0, 0)),
            pl.BlockSpec((1, O, 1), lambda b, t: (b, 0, 0)),
        ],
        out_shape=[
            jax.ShapeDtypeStruct((B, T, 1, TP), jnp.float32),
            jax.ShapeDtypeStruct((B, T, 1, TP), jnp.int32),
            jax.ShapeDtypeStruct((B, O, 1), jnp.int32),
        ],
        scratch_shapes=[
            pltpu.VMEM((O, 1), jnp.float32),
            pltpu.VMEM((O, 1), jnp.int32),
        ],
    )(priors_t, targets)

    rows, parts = pl.pallas_call(
        functools.partial(_loss_kernel, TP=TP, O=O),
        grid=(B, T),
        in_specs=[
            pl.BlockSpec((1, TP, 4), lambda b, t: (b, t, 0)),
            pl.BlockSpec((1, TP, 2), lambda b, t: (b, t, 0)),
            pl.BlockSpec((1, TP, 10), lambda b, t: (b, t, 0)),
            pl.BlockSpec((TP, 4), lambda b, t: (t, 0)),
            pl.BlockSpec((1, O, 15), lambda b, t: (b, 0, 0)),
            pl.BlockSpec((1, 1, 1, TP), lambda b, t: (b, t, 0, 0)),
            pl.BlockSpec((1, 1, 1, TP), lambda b, t: (b, t, 0, 0)),
            pl.BlockSpec((1, O, 1), lambda b, t: (b, 0, 0)),
        ],
        out_specs=[
            pl.BlockSpec((1, 1, TP, 1), lambda b, t: (b, t, 0, 0)),
            pl.BlockSpec((1, 1, 1, 8), lambda b, t: (b, t, 0, 0)),
        ],
        out_shape=[
            jax.ShapeDtypeStruct((B, T, TP, 1), jnp.float32),
            jax.ShapeDtypeStruct((B, T, 1, 8), jnp.float32),
        ],
    )(loc_data, conf_data, loc_five_data, priors, targets, bto, bti, bpi)

    rows_r = rows.reshape(B, P // 128, 128)
    topk = pl.pallas_call(
        functools.partial(_topk_kernel, P=P),
        grid=(B,),
        in_specs=[
            pl.BlockSpec((1, P // 128, 128), lambda b: (b, 0, 0)),
            pl.BlockSpec((1, T, 1, 8), lambda b: (b, 0, 0, 0)),
        ],
        out_specs=pl.BlockSpec((1, 1, 8), lambda b: (b, 0, 0)),
        out_shape=jax.ShapeDtypeStruct((B, 1, 8), jnp.float32),
    )(rows_r, parts)

    loss_l = jnp.sum(parts[..., 0])
    loss_coords = jnp.sum(parts[..., 1])
    pos_ce = jnp.sum(parts[..., 2])
    np_tot = jnp.sum(parts[..., 3])
    topk_tot = jnp.sum(topk[:, 0, 0])
    n = jnp.maximum(np_tot, 1.0)
    return (loss_l / n, (pos_ce + topk_tot) / n, loss_coords / n)


# speedup vs baseline: 46.0572x; 5.6831x over previous
import functools

import jax
import jax.numpy as jnp
from jax.experimental import pallas as pl
from jax.experimental.pallas import tpu as pltpu

_THRESHOLD = 0.35
_NEGPOS_RATIO = 7
_VAR0 = 0.1
_VAR1 = 0.2


def _match_kernel(pt_ref, tg_ref, bto_ref, bti_ref, bpi_ref, sv_ref, si_ref,
                  *, TP, T, O):
    t = pl.program_id(1)

    @pl.when(t == 0)
    def _init():
        sv_ref[...] = jnp.full((O, 1), -1.0, jnp.float32)
        si_ref[...] = jnp.zeros((O, 1), jnp.int32)

    px = pt_ref[0:1, :]
    py = pt_ref[1:2, :]
    pw = pt_ref[2:3, :]
    ph = pt_ref[3:4, :]
    bx1 = px - pw / 2.0
    by1 = py - ph / 2.0
    bx2 = px + pw / 2.0
    by2 = py + ph / 2.0
    area_b = (bx2 - bx1) * (by2 - by1)

    tb = tg_ref[0]
    tx1 = tb[:, 0:1]
    ty1 = tb[:, 1:2]
    tx2 = tb[:, 2:3]
    ty2 = tb[:, 3:4]
    area_a = (tx2 - tx1) * (ty2 - ty1)

    iw = jnp.maximum(jnp.minimum(tx2, bx2) - jnp.maximum(tx1, bx1), 0.0)
    ih = jnp.maximum(jnp.minimum(ty2, by2) - jnp.maximum(ty1, by1), 0.0)
    inter = iw * ih
    iou = inter / (area_a + area_b - inter)

    io_o = jax.lax.broadcasted_iota(jnp.int32, (O, 1), 0)
    io_p = jax.lax.broadcasted_iota(jnp.int32, (1, TP), 1)

    pm = jnp.max(iou, axis=0, keepdims=True)
    pidx = jnp.min(jnp.where(iou == pm, io_o, O), axis=0, keepdims=True)
    bto_ref[0, 0] = pm
    bti_ref[0, 0] = pidx

    tv = jnp.max(iou, axis=1, keepdims=True)
    tl = jnp.min(jnp.where(iou == tv, io_p, TP), axis=1, keepdims=True)
    gidx = t * TP + tl
    sv = sv_ref[...]
    si = si_ref[...]
    upd = tv > sv
    new_sv = jnp.where(upd, tv, sv)
    new_si = jnp.where(upd, gidx, si)
    sv_ref[...] = new_sv
    si_ref[...] = new_si
    bpi_ref[0] = new_si


def _smooth_l1(d):
    ad = jnp.abs(d)
    return jnp.where(ad < 1.0, 0.5 * d * d, ad - 0.5)


def _loss_kernel(loc_ref, conf_ref, five_ref, pt_ref, tg_ref, bto_ref,
                 bti_ref, bpi_ref, rows_ref, part_ref, *, TP, O):
    t = pl.program_id(1)

    bto = bto_ref[0, 0]
    bti = bti_ref[0, 0]
    bpi = bpi_ref[0]

    io_o = jax.lax.broadcasted_iota(jnp.int32, (O, 1), 0)
    gp = t * TP + jax.lax.broadcasted_iota(jnp.int32, (1, TP), 1)

    match = bpi == gp
    forced = jnp.max(match.astype(jnp.int32), axis=0, keepdims=True) > 0
    fidx = jnp.max(jnp.where(match, io_o, -1), axis=0, keepdims=True)
    idx_f = jnp.where(forced, fidx, bti)
    pos_row = jnp.logical_or(forced, bto >= _THRESHOLD)
    posf = pos_row.astype(jnp.float32)

    oh = (io_o == idx_f).astype(jnp.float32)
    tb = tg_ref[0]
    table = jnp.concatenate([tb[:, 0:4], tb[:, 5:15]], axis=1)
    matched = jax.lax.dot_general(
        table, oh, (((0,), (0,)), ((), ())),
        preferred_element_type=jnp.float32)

    pcxy = pt_ref[0:2, :]
    pwh = pt_ref[2:4, :]

    m12 = matched[0:2, :]
    m34 = matched[2:4, :]
    g_cxcy = ((m12 + m34) / 2.0 - pcxy) / (_VAR0 * pwh)
    g_wh = jnp.log((m34 - m12) / pwh) / _VAR1
    loc = loc_ref[0]
    sl = _smooth_l1(loc[0:2, :] - g_cxcy) + _smooth_l1(loc[2:4, :] - g_wh)
    loss_l = jnp.sum(sl * posf)

    ml = matched[4:14, :]
    pc_rep = jnp.concatenate([pcxy] * 5, axis=0)
    pwh_rep = jnp.concatenate([pwh] * 5, axis=0)
    g_lm = (ml - pc_rep) / (_VAR0 * pwh_rep)
    dlm = five_ref[0] - g_lm
    loss_coords = jnp.sum(dlm * dlm * posf)

    c = conf_ref[0]
    c0 = c[0:1, :]
    c1 = c[1:2, :]
    mx = jnp.maximum(c0, c1)
    lse = mx + jnp.log(jnp.exp(c0 - mx) + jnp.exp(c1 - mx))
    ce = lse - jnp.where(pos_row, c1, c0)
    pos_ce = jnp.sum(jnp.where(pos_row, ce, 0.0))
    negloss = jnp.where(pos_row, 0.0, lse - c0)
    rows_ref[0, 0] = negloss
    npos = jnp.sum(posf)

    lane = jax.lax.broadcasted_iota(jnp.int32, (1, 8), 1)
    vec = (jnp.where(lane == 0, loss_l, 0.0)
           + jnp.where(lane == 1, loss_coords, 0.0)
           + jnp.where(lane == 2, pos_ce, 0.0)
           + jnp.where(lane == 3, npos, 0.0))
    part_ref[0, 0] = vec


def _topk_kernel(rows_ref, part_ref, out_ref, *, P):
    v = rows_ref[0]
    bits = jax.lax.bitcast_convert_type(v, jnp.int32)
    npos = jnp.sum(part_ref[0, :, 0, 3])
    k = jnp.minimum(_NEGPOS_RATIO * npos.astype(jnp.int32), P - 1)

    def body(_, carry):
        lo, hi = carry
        mid = lo + (hi - lo) // 2
        cnt = jnp.sum((bits >= mid).astype(jnp.int32))
        good = cnt >= k
        return (jnp.where(good, mid, lo), jnp.where(good, hi, mid))

    lo, _ = jax.lax.fori_loop(
        0, 31, body, (jnp.int32(0), jnp.int32(0x7F800000)))
    tf = jax.lax.bitcast_convert_type(lo, jnp.float32)
    gt = bits > lo
    cnt_gt = jnp.sum(gt.astype(jnp.float32))
    sum_gt = jnp.sum(jnp.where(gt, v, 0.0))
    topk = sum_gt + (k.astype(jnp.float32) - cnt_gt) * tf
    topk = jnp.where(k >= 1, topk, 0.0)
    lane = jax.lax.broadcasted_iota(jnp.int32, (1, 8), 1)
    out_ref[0] = jnp.where(lane == 0, topk, 0.0)


@jax.jit
def kernel(loc_data, conf_data, loc_five_data, priors, targets):
    B, P, _ = loc_data.shape
    O = targets.shape[1]
    TP = 2048 if P % 2048 == 0 else P
    T = P // TP

    priors_t = priors.T

    bto, bti, bpi = pl.pallas_call(
        functools.partial(_match_kernel, TP=TP, T=T, O=O),
        grid=(B, T),
        in_specs=[
            pl.BlockSpec((4, TP), lambda b, t: (0, t)),
            pl.BlockSpec((1, O, 15), lambda b, t: (b, 0, 0)),
        ],
        out_specs=[
            pl.BlockSpec((1, 1, 1, TP), lambda b, t: (b, t, 0, 0)),
            pl.BlockSpec((1, 1, 1, TP), lambda b, t: (b, t, 0, 0)),
            pl.BlockSpec((1, O, 1), lambda b, t: (b, 0, 0)),
        ],
        out_shape=[
            jax.ShapeDtypeStruct((B, T, 1, TP), jnp.float32),
            jax.ShapeDtypeStruct((B, T, 1, TP), jnp.int32),
            jax.ShapeDtypeStruct((B, O, 1), jnp.int32),
        ],
        scratch_shapes=[
            pltpu.VMEM((O, 1), jnp.float32),
            pltpu.VMEM((O, 1), jnp.int32),
        ],
    )(priors_t, targets)

    loc_t = loc_data.transpose(0, 2, 1)
    conf_t = conf_data.transpose(0, 2, 1)
    five_t = loc_five_data.transpose(0, 2, 1)

    rows, parts = pl.pallas_call(
        functools.partial(_loss_kernel, TP=TP, O=O),
        grid=(B, T),
        in_specs=[
            pl.BlockSpec((1, 4, TP), lambda b, t: (b, 0, t)),
            pl.BlockSpec((1, 2, TP), lambda b, t: (b, 0, t)),
            pl.BlockSpec((1, 10, TP), lambda b, t: (b, 0, t)),
            pl.BlockSpec((4, TP), lambda b, t: (0, t)),
            pl.BlockSpec((1, O, 15), lambda b, t: (b, 0, 0)),
            pl.BlockSpec((1, 1, 1, TP), lambda b, t: (b, t, 0, 0)),
            pl.BlockSpec((1, 1, 1, TP), lambda b, t: (b, t, 0, 0)),
            pl.BlockSpec((1, O, 1), lambda b, t: (b, 0, 0)),
        ],
        out_specs=[
            pl.BlockSpec((1, 1, 1, TP), lambda b, t: (b, t, 0, 0)),
            pl.BlockSpec((1, 1, 1, 8), lambda b, t: (b, t, 0, 0)),
        ],
        out_shape=[
            jax.ShapeDtypeStruct((B, T, 1, TP), jnp.float32),
            jax.ShapeDtypeStruct((B, T, 1, 8), jnp.float32),
        ],
    )(loc_t, conf_t, five_t, priors_t, targets, bto, bti, bpi)

    rows_r = rows.reshape(B, P // 128, 128)
    topk = pl.pallas_call(
        functools.partial(_topk_kernel, P=P),
        grid=(B,),
        in_specs=[
            pl.BlockSpec((1, P // 128, 128), lambda b: (b, 0, 0)),
            pl.BlockSpec((1, T, 1, 8), lambda b: (b, 0, 0, 0)),
        ],
        out_specs=pl.BlockSpec((1, 1, 8), lambda b: (b, 0, 0)),
        out_shape=jax.ShapeDtypeStruct((B, 1, 8), jnp.float32),
    )(rows_r, parts)

    loss_l = jnp.sum(parts[..., 0])
    loss_coords = jnp.sum(parts[..., 1])
    pos_ce = jnp.sum(parts[..., 2])
    np_tot = jnp.sum(parts[..., 3])
    topk_tot = jnp.sum(topk[:, 0, 0])
    n = jnp.maximum(np_tot, 1.0)
    return (loss_l / n, (pos_ce + topk_tot) / n, loss_coords / n)


# speedup vs baseline: 83.2506x; 1.8075x over previous
import functools

import jax
import jax.numpy as jnp
from jax.experimental import pallas as pl
from jax.experimental.pallas import tpu as pltpu

_THRESHOLD = 0.35
_NEGPOS_RATIO = 7
_VAR0 = 0.1
_VAR1 = 0.2


def _match_kernel(pt_ref, tg_ref, bto_ref, bti_ref, bpi_ref, sv_ref, si_ref,
                  *, TP, T, O):
    t = pl.program_id(1)

    @pl.when(t == 0)
    def _init():
        sv_ref[...] = jnp.full((O, 1), -1.0, jnp.float32)
        si_ref[...] = jnp.zeros((O, 1), jnp.int32)

    px = pt_ref[0:1, :]
    py = pt_ref[1:2, :]
    pw = pt_ref[2:3, :]
    ph = pt_ref[3:4, :]
    bx1 = px - pw / 2.0
    by1 = py - ph / 2.0
    bx2 = px + pw / 2.0
    by2 = py + ph / 2.0
    area_b = (bx2 - bx1) * (by2 - by1)

    tb = tg_ref[0]
    tx1 = tb[:, 0:1]
    ty1 = tb[:, 1:2]
    tx2 = tb[:, 2:3]
    ty2 = tb[:, 3:4]
    area_a = (tx2 - tx1) * (ty2 - ty1)

    iw = jnp.maximum(jnp.minimum(tx2, bx2) - jnp.maximum(tx1, bx1), 0.0)
    ih = jnp.maximum(jnp.minimum(ty2, by2) - jnp.maximum(ty1, by1), 0.0)
    inter = iw * ih
    iou = inter / (area_a + area_b - inter)

    io_o = jax.lax.broadcasted_iota(jnp.int32, (O, 1), 0)
    io_p = jax.lax.broadcasted_iota(jnp.int32, (1, TP), 1)

    pm = jnp.max(iou, axis=0, keepdims=True)
    pidx = jnp.min(jnp.where(iou == pm, io_o, O), axis=0, keepdims=True)
    bto_ref[0, 0] = pm
    bti_ref[0, 0] = pidx

    tv = jnp.max(iou, axis=1, keepdims=True)
    tl = jnp.min(jnp.where(iou == tv, io_p, TP), axis=1, keepdims=True)
    gidx = t * TP + tl
    sv = sv_ref[...]
    si = si_ref[...]
    upd = tv > sv
    new_sv = jnp.where(upd, tv, sv)
    new_si = jnp.where(upd, gidx, si)
    sv_ref[...] = new_sv
    si_ref[...] = new_si
    bpi_ref[0] = new_si


def _smooth_l1(d):
    ad = jnp.abs(d)
    return jnp.where(ad < 1.0, 0.5 * d * d, ad - 0.5)


def _loss_kernel(loc_ref, conf_ref, five_ref, pt_ref, tg_ref, bto_ref,
                 bti_ref, bpi_ref, rows_ref, part_ref, *, TP, O):
    t = pl.program_id(1)

    bto = bto_ref[0, 0]
    bti = bti_ref[0, 0]
    bpi = bpi_ref[0]

    io_o = jax.lax.broadcasted_iota(jnp.int32, (O, 1), 0)
    gp = t * TP + jax.lax.broadcasted_iota(jnp.int32, (1, TP), 1)

    match = bpi == gp
    forced = jnp.max(match.astype(jnp.int32), axis=0, keepdims=True) > 0
    fidx = jnp.max(jnp.where(match, io_o, -1), axis=0, keepdims=True)
    idx_f = jnp.where(forced, fidx, bti)
    pos_row = jnp.logical_or(forced, bto >= _THRESHOLD)
    posf = pos_row.astype(jnp.float32)

    oh = (io_o == idx_f).astype(jnp.float32)
    tb = tg_ref[0]
    table = jnp.concatenate([tb[:, 0:4], tb[:, 5:15]], axis=1)
    matched = jax.lax.dot_general(
        table, oh, (((0,), (0,)), ((), ())),
        preferred_element_type=jnp.float32)

    pcxy = pt_ref[0:2, :]
    pwh = pt_ref[2:4, :]

    m12 = matched[0:2, :]
    m34 = matched[2:4, :]
    g_cxcy = ((m12 + m34) / 2.0 - pcxy) / (_VAR0 * pwh)
    g_wh = jnp.log((m34 - m12) / pwh) / _VAR1
    loc = loc_ref[0]
    sl = _smooth_l1(loc[0:2, :] - g_cxcy) + _smooth_l1(loc[2:4, :] - g_wh)
    loss_l = jnp.sum(sl * posf)

    ml = matched[4:14, :]
    pc_rep = jnp.concatenate([pcxy] * 5, axis=0)
    pwh_rep = jnp.concatenate([pwh] * 5, axis=0)
    g_lm = (ml - pc_rep) / (_VAR0 * pwh_rep)
    dlm = five_ref[0] - g_lm
    loss_coords = jnp.sum(dlm * dlm * posf)

    c = conf_ref[0]
    c0 = c[0:1, :]
    c1 = c[1:2, :]
    mx = jnp.maximum(c0, c1)
    lse = mx + jnp.log(jnp.exp(c0 - mx) + jnp.exp(c1 - mx))
    ce = lse - jnp.where(pos_row, c1, c0)
    pos_ce = jnp.sum(jnp.where(pos_row, ce, 0.0))
    negloss = jnp.where(pos_row, 0.0, lse - c0)
    rows_ref[0, 0] = negloss
    npos = jnp.sum(posf)

    lane = jax.lax.broadcasted_iota(jnp.int32, (1, 8), 1)
    vec = (jnp.where(lane == 0, loss_l, 0.0)
           + jnp.where(lane == 1, loss_coords, 0.0)
           + jnp.where(lane == 2, pos_ce, 0.0)
           + jnp.where(lane == 3, npos, 0.0))
    part_ref[0, 0] = vec


def _topk_kernel(rows_ref, part_ref, out_ref, *, P, B):
    v = rows_ref[...]
    bits = jax.lax.bitcast_convert_type(v, jnp.int32)
    npos = jnp.sum(part_ref[:, :, 0, 3], axis=1, keepdims=True)[..., None]
    k = jnp.minimum(_NEGPOS_RATIO * npos.astype(jnp.int32), P - 1)

    def body(_, carry):
        lo, hi = carry
        mid = lo + (hi - lo) // 2
        cnt = jnp.sum((bits >= mid).astype(jnp.int32), axis=(1, 2),
                      keepdims=True)
        good = cnt >= k
        return (jnp.where(good, mid, lo), jnp.where(good, hi, mid))

    z = jnp.zeros_like(k)
    lo, _ = jax.lax.fori_loop(0, 31, body, (z, z + 0x7F800000))
    tf = jax.lax.bitcast_convert_type(lo, jnp.float32)
    gt = bits > lo
    cnt_gt = jnp.sum(gt.astype(jnp.float32), axis=(1, 2), keepdims=True)
    sum_gt = jnp.sum(jnp.where(gt, v, 0.0), axis=(1, 2), keepdims=True)
    topk = sum_gt + (k.astype(jnp.float32) - cnt_gt) * tf
    topk = jnp.where(k >= 1, topk, 0.0)
    lane = jax.lax.broadcasted_iota(jnp.int32, (B, 1, 8), 2)
    out_ref[...] = jnp.where(lane == 0, topk, 0.0)


@jax.jit
def kernel(loc_data, conf_data, loc_five_data, priors, targets):
    B, P, _ = loc_data.shape
    O = targets.shape[1]
    TP = 4096 if P % 4096 == 0 else P
    T = P // TP

    priors_t = priors.T

    bto, bti, bpi = pl.pallas_call(
        functools.partial(_match_kernel, TP=TP, T=T, O=O),
        grid=(B, T),
        in_specs=[
            pl.BlockSpec((4, TP), lambda b, t: (0, t)),
            pl.BlockSpec((1, O, 15), lambda b, t: (b, 0, 0)),
        ],
        out_specs=[
            pl.BlockSpec((1, 1, 1, TP), lambda b, t: (b, t, 0, 0)),
            pl.BlockSpec((1, 1, 1, TP), lambda b, t: (b, t, 0, 0)),
            pl.BlockSpec((1, O, 1), lambda b, t: (b, 0, 0)),
        ],
        out_shape=[
            jax.ShapeDtypeStruct((B, T, 1, TP), jnp.float32),
            jax.ShapeDtypeStruct((B, T, 1, TP), jnp.int32),
            jax.ShapeDtypeStruct((B, O, 1), jnp.int32),
        ],
        scratch_shapes=[
            pltpu.VMEM((O, 1), jnp.float32),
            pltpu.VMEM((O, 1), jnp.int32),
        ],
    )(priors_t, targets)

    loc_t = loc_data.transpose(0, 2, 1)
    conf_t = conf_data.transpose(0, 2, 1)
    five_t = loc_five_data.transpose(0, 2, 1)

    rows, parts = pl.pallas_call(
        functools.partial(_loss_kernel, TP=TP, O=O),
        grid=(B, T),
        in_specs=[
            pl.BlockSpec((1, 4, TP), lambda b, t: (b, 0, t)),
            pl.BlockSpec((1, 2, TP), lambda b, t: (b, 0, t)),
            pl.BlockSpec((1, 10, TP), lambda b, t: (b, 0, t)),
            pl.BlockSpec((4, TP), lambda b, t: (0, t)),
            pl.BlockSpec((1, O, 15), lambda b, t: (b, 0, 0)),
            pl.BlockSpec((1, 1, 1, TP), lambda b, t: (b, t, 0, 0)),
            pl.BlockSpec((1, 1, 1, TP), lambda b, t: (b, t, 0, 0)),
            pl.BlockSpec((1, O, 1), lambda b, t: (b, 0, 0)),
        ],
        out_specs=[
            pl.BlockSpec((1, 1, 1, TP), lambda b, t: (b, t, 0, 0)),
            pl.BlockSpec((1, 1, 1, 8), lambda b, t: (b, t, 0, 0)),
        ],
        out_shape=[
            jax.ShapeDtypeStruct((B, T, 1, TP), jnp.float32),
            jax.ShapeDtypeStruct((B, T, 1, 8), jnp.float32),
        ],
    )(loc_t, conf_t, five_t, priors_t, targets, bto, bti, bpi)

    rows_r = rows.reshape(B, P // 128, 128)
    topk = pl.pallas_call(
        functools.partial(_topk_kernel, P=P, B=B),
        grid=(1,),
        in_specs=[
            pl.BlockSpec((B, P // 128, 128), lambda i: (0, 0, 0)),
            pl.BlockSpec((B, T, 1, 8), lambda i: (0, 0, 0, 0)),
        ],
        out_specs=pl.BlockSpec((B, 1, 8), lambda i: (0, 0, 0)),
        out_shape=jax.ShapeDtypeStruct((B, 1, 8), jnp.float32),
    )(rows_r, parts)

    loss_l = jnp.sum(parts[..., 0])
    loss_coords = jnp.sum(parts[..., 1])
    pos_ce = jnp.sum(parts[..., 2])
    np_tot = jnp.sum(parts[..., 3])
    topk_tot = jnp.sum(topk[:, 0, 0])
    n = jnp.maximum(np_tot, 1.0)
    return (loss_l / n, (pos_ce + topk_tot) / n, loss_coords / n)


# speedup vs baseline: 101.7712x; 1.2225x over previous
import functools

import jax
import jax.numpy as jnp
from jax.experimental import pallas as pl
from jax.experimental.pallas import tpu as pltpu

_THRESHOLD = 0.35
_NEGPOS_RATIO = 7
_VAR0 = 0.1
_VAR1 = 0.2


def _match_kernel(pt_ref, tg_ref, bto_ref, bti_ref, bpi_ref, sv_ref, si_ref,
                  *, TP, T, O):
    t = pl.program_id(1)

    @pl.when(t == 0)
    def _init():
        sv_ref[...] = jnp.full((O, 1), -1.0, jnp.float32)
        si_ref[...] = jnp.zeros((O, 1), jnp.int32)

    px = pt_ref[0:1, :]
    py = pt_ref[1:2, :]
    pw = pt_ref[2:3, :]
    ph = pt_ref[3:4, :]
    bx1 = px - pw / 2.0
    by1 = py - ph / 2.0
    bx2 = px + pw / 2.0
    by2 = py + ph / 2.0
    area_b = (bx2 - bx1) * (by2 - by1)

    tb = tg_ref[0]
    tx1 = tb[:, 0:1]
    ty1 = tb[:, 1:2]
    tx2 = tb[:, 2:3]
    ty2 = tb[:, 3:4]
    area_a = (tx2 - tx1) * (ty2 - ty1)

    iw = jnp.maximum(jnp.minimum(tx2, bx2) - jnp.maximum(tx1, bx1), 0.0)
    ih = jnp.maximum(jnp.minimum(ty2, by2) - jnp.maximum(ty1, by1), 0.0)
    inter = iw * ih
    iou = inter / (area_a + area_b - inter)

    io_o = jax.lax.broadcasted_iota(jnp.int32, (O, 1), 0)
    io_p = jax.lax.broadcasted_iota(jnp.int32, (1, TP), 1)

    pm = jnp.max(iou, axis=0, keepdims=True)
    pidx = jnp.min(jnp.where(iou == pm, io_o, O), axis=0, keepdims=True)
    bto_ref[0, 0] = pm
    bti_ref[0, 0] = pidx

    tv = jnp.max(iou, axis=1, keepdims=True)
    tl = jnp.min(jnp.where(iou == tv, io_p, TP), axis=1, keepdims=True)
    gidx = t * TP + tl
    sv = sv_ref[...]
    si = si_ref[...]
    upd = tv > sv
    new_sv = jnp.where(upd, tv, sv)
    new_si = jnp.where(upd, gidx, si)
    sv_ref[...] = new_sv
    si_ref[...] = new_si
    bpi_ref[0] = new_si


def _smooth_l1(d):
    ad = jnp.abs(d)
    return jnp.where(ad < 1.0, 0.5 * d * d, ad - 0.5)


def _loss_kernel(loc_ref, conf_ref, five_ref, pt_ref, tg_ref, bto_ref,
                 bti_ref, bpi_ref, rows_ref, part_ref, *, TP, O):
    t = pl.program_id(1)

    bto = bto_ref[0, 0]
    bti = bti_ref[0, 0]
    bpi = bpi_ref[0]

    io_o = jax.lax.broadcasted_iota(jnp.int32, (O, 1), 0)
    gp = t * TP + jax.lax.broadcasted_iota(jnp.int32, (1, TP), 1)

    match = bpi == gp
    forced = jnp.max(match.astype(jnp.int32), axis=0, keepdims=True) > 0
    fidx = jnp.max(jnp.where(match, io_o, -1), axis=0, keepdims=True)
    idx_f = jnp.where(forced, fidx, bti)
    pos_row = jnp.logical_or(forced, bto >= _THRESHOLD)
    posf = pos_row.astype(jnp.float32)

    oh = (io_o == idx_f).astype(jnp.float32)
    tb = tg_ref[0]
    table = jnp.concatenate([tb[:, 0:4], tb[:, 5:15]], axis=1)
    matched = jax.lax.dot_general(
        table, oh, (((0,), (0,)), ((), ())),
        preferred_element_type=jnp.float32)

    pcxy = pt_ref[0:2, :]
    pwh = pt_ref[2:4, :]

    m12 = matched[0:2, :]
    m34 = matched[2:4, :]
    g_cxcy = ((m12 + m34) / 2.0 - pcxy) / (_VAR0 * pwh)
    g_wh = jnp.log((m34 - m12) / pwh) / _VAR1
    loc = loc_ref[0]
    sl = _smooth_l1(loc[0:2, :] - g_cxcy) + _smooth_l1(loc[2:4, :] - g_wh)
    loss_l = jnp.sum(sl * posf)

    ml = matched[4:14, :]
    pc_rep = jnp.concatenate([pcxy] * 5, axis=0)
    pwh_rep = jnp.concatenate([pwh] * 5, axis=0)
    g_lm = (ml - pc_rep) / (_VAR0 * pwh_rep)
    dlm = five_ref[0] - g_lm
    loss_coords = jnp.sum(dlm * dlm * posf)

    c = conf_ref[0]
    c0 = c[0:1, :]
    c1 = c[1:2, :]
    mx = jnp.maximum(c0, c1)
    lse = mx + jnp.log(jnp.exp(c0 - mx) + jnp.exp(c1 - mx))
    ce = lse - jnp.where(pos_row, c1, c0)
    pos_ce = jnp.sum(jnp.where(pos_row, ce, 0.0))
    negloss = jnp.where(pos_row, 0.0, lse - c0)
    rows_ref[0, 0] = negloss
    npos = jnp.sum(posf)

    lane = jax.lax.broadcasted_iota(jnp.int32, (1, 8), 1)
    vec = (jnp.where(lane == 0, loss_l, 0.0)
           + jnp.where(lane == 1, loss_coords, 0.0)
           + jnp.where(lane == 2, pos_ce, 0.0)
           + jnp.where(lane == 3, npos, 0.0))
    part_ref[0, 0] = vec


def _topk_kernel(rows_ref, part_ref, out_ref, *, P, B):
    v = rows_ref[...]
    bits = jax.lax.bitcast_convert_type(v, jnp.int32)
    npos = jnp.sum(part_ref[:, :, 0, 3], axis=1, keepdims=True)[..., None]
    k = jnp.minimum(_NEGPOS_RATIO * npos.astype(jnp.int32), P - 1)

    def body(_, carry):
        lo, hi = carry
        mid = lo + (hi - lo) // 2
        cnt = jnp.sum((bits >= mid).astype(jnp.int32), axis=(1, 2),
                      keepdims=True)
        good = cnt >= k
        return (jnp.where(good, mid, lo), jnp.where(good, hi, mid))

    z = jnp.zeros_like(k)
    lo, _ = jax.lax.fori_loop(0, 31, body, (z, z + 0x7F800000))
    tf = jax.lax.bitcast_convert_type(lo, jnp.float32)
    gt = bits > lo
    cnt_gt = jnp.sum(gt.astype(jnp.float32), axis=(1, 2), keepdims=True)
    sum_gt = jnp.sum(jnp.where(gt, v, 0.0), axis=(1, 2), keepdims=True)
    topk = sum_gt + (k.astype(jnp.float32) - cnt_gt) * tf
    topk = jnp.where(k >= 1, topk, 0.0)
    lane = jax.lax.broadcasted_iota(jnp.int32, (B, 1, 8), 2)
    out_ref[...] = jnp.where(lane == 0, topk, 0.0)


@jax.jit
def kernel(loc_data, conf_data, loc_five_data, priors, targets):
    B, P, _ = loc_data.shape
    O = targets.shape[1]
    TP = 8192 if P % 8192 == 0 else P
    T = P // TP

    priors_t = priors.T

    bto, bti, bpi = pl.pallas_call(
        functools.partial(_match_kernel, TP=TP, T=T, O=O),
        grid=(B, T),
        in_specs=[
            pl.BlockSpec((4, TP), lambda b, t: (0, t)),
            pl.BlockSpec((1, O, 15), lambda b, t: (b, 0, 0)),
        ],
        out_specs=[
            pl.BlockSpec((1, 1, 1, TP), lambda b, t: (b, t, 0, 0)),
            pl.BlockSpec((1, 1, 1, TP), lambda b, t: (b, t, 0, 0)),
            pl.BlockSpec((1, O, 1), lambda b, t: (b, 0, 0)),
        ],
        out_shape=[
            jax.ShapeDtypeStruct((B, T, 1, TP), jnp.float32),
            jax.ShapeDtypeStruct((B, T, 1, TP), jnp.int32),
            jax.ShapeDtypeStruct((B, O, 1), jnp.int32),
        ],
        scratch_shapes=[
            pltpu.VMEM((O, 1), jnp.float32),
            pltpu.VMEM((O, 1), jnp.int32),
        ],
    )(priors_t, targets)

    loc_t = loc_data.transpose(0, 2, 1)
    conf_t = conf_data.transpose(0, 2, 1)
    five_t = loc_five_data.transpose(0, 2, 1)

    rows, parts = pl.pallas_call(
        functools.partial(_loss_kernel, TP=TP, O=O),
        grid=(B, T),
        in_specs=[
            pl.BlockSpec((1, 4, TP), lambda b, t: (b, 0, t)),
            pl.BlockSpec((1, 2, TP), lambda b, t: (b, 0, t)),
            pl.BlockSpec((1, 10, TP), lambda b, t: (b, 0, t)),
            pl.BlockSpec((4, TP), lambda b, t: (0, t)),
            pl.BlockSpec((1, O, 15), lambda b, t: (b, 0, 0)),
            pl.BlockSpec((1, 1, 1, TP), lambda b, t: (b, t, 0, 0)),
            pl.BlockSpec((1, 1, 1, TP), lambda b, t: (b, t, 0, 0)),
            pl.BlockSpec((1, O, 1), lambda b, t: (b, 0, 0)),
        ],
        out_specs=[
            pl.BlockSpec((1, 1, 1, TP), lambda b, t: (b, t, 0, 0)),
            pl.BlockSpec((1, 1, 1, 8), lambda b, t: (b, t, 0, 0)),
        ],
        out_shape=[
            jax.ShapeDtypeStruct((B, T, 1, TP), jnp.float32),
            jax.ShapeDtypeStruct((B, T, 1, 8), jnp.float32),
        ],
    )(loc_t, conf_t, five_t, priors_t, targets, bto, bti, bpi)

    rows_r = rows.reshape(B, P // 128, 128)
    topk = pl.pallas_call(
        functools.partial(_topk_kernel, P=P, B=B),
        grid=(1,),
        in_specs=[
            pl.BlockSpec((B, P // 128, 128), lambda i: (0, 0, 0)),
            pl.BlockSpec((B, T, 1, 8), lambda i: (0, 0, 0, 0)),
        ],
        out_specs=pl.BlockSpec((B, 1, 8), lambda i: (0, 0, 0)),
        out_shape=jax.ShapeDtypeStruct((B, 1, 8), jnp.float32),
    )(rows_r, parts)

    loss_l = jnp.sum(parts[..., 0])
    loss_coords = jnp.sum(parts[..., 1])
    pos_ce = jnp.sum(parts[..., 2])
    np_tot = jnp.sum(parts[..., 3])
    topk_tot = jnp.sum(topk[:, 0, 0])
    n = jnp.maximum(np_tot, 1.0)
    return (loss_l / n, (pos_ce + topk_tot) / n, loss_coords / n)


# speedup vs baseline: 108.2501x; 1.0637x over previous
import functools

import jax
import jax.numpy as jnp
from jax.experimental import pallas as pl
from jax.experimental.pallas import tpu as pltpu

_THRESHOLD = 0.35
_NEGPOS_RATIO = 7
_VAR0 = 0.1
_VAR1 = 0.2


def _match_kernel(pt_ref, tg_ref, bto_ref, bti_ref, bpi_ref, sv_ref, si_ref,
                  *, TP, T, O):
    t = pl.program_id(1)

    @pl.when(t == 0)
    def _init():
        sv_ref[...] = jnp.full((O, 1), -1.0, jnp.float32)
        si_ref[...] = jnp.zeros((O, 1), jnp.int32)

    px = pt_ref[0:1, :]
    py = pt_ref[1:2, :]
    pw = pt_ref[2:3, :]
    ph = pt_ref[3:4, :]
    bx1 = px - pw / 2.0
    by1 = py - ph / 2.0
    bx2 = px + pw / 2.0
    by2 = py + ph / 2.0
    area_b = (bx2 - bx1) * (by2 - by1)

    tb = tg_ref[0]
    tx1 = tb[:, 0:1]
    ty1 = tb[:, 1:2]
    tx2 = tb[:, 2:3]
    ty2 = tb[:, 3:4]
    area_a = (tx2 - tx1) * (ty2 - ty1)

    iw = jnp.maximum(jnp.minimum(tx2, bx2) - jnp.maximum(tx1, bx1), 0.0)
    ih = jnp.maximum(jnp.minimum(ty2, by2) - jnp.maximum(ty1, by1), 0.0)
    inter = iw * ih
    iou = inter / (area_a + area_b - inter)

    io_o = jax.lax.broadcasted_iota(jnp.int32, (O, 1), 0)
    io_p = jax.lax.broadcasted_iota(jnp.int32, (1, TP), 1)

    pm = jnp.max(iou, axis=0, keepdims=True)
    pidx = jnp.min(jnp.where(iou == pm, io_o, O), axis=0, keepdims=True)
    bto_ref[0, 0] = pm
    bti_ref[0, 0] = pidx

    tv = jnp.max(iou, axis=1, keepdims=True)
    tl = jnp.min(jnp.where(iou == tv, io_p, TP), axis=1, keepdims=True)
    gidx = t * TP + tl
    sv = sv_ref[...]
    si = si_ref[...]
    upd = tv > sv
    new_sv = jnp.where(upd, tv, sv)
    new_si = jnp.where(upd, gidx, si)
    sv_ref[...] = new_sv
    si_ref[...] = new_si
    bpi_ref[0] = new_si


def _smooth_l1(d):
    ad = jnp.abs(d)
    return jnp.where(ad < 1.0, 0.5 * d * d, ad - 0.5)


def _loss_kernel(loc_ref, conf_ref, five_ref, pt_ref, tg_ref, bto_ref,
                 bti_ref, bpi_ref, rows_ref, part_ref, *, TP, O):
    t = pl.program_id(1)

    bto = bto_ref[0, 0]
    bti = bti_ref[0, 0]
    bpi = bpi_ref[0]

    io_o = jax.lax.broadcasted_iota(jnp.int32, (O, 1), 0)
    gp = t * TP + jax.lax.broadcasted_iota(jnp.int32, (1, TP), 1)

    match = bpi == gp
    forced = jnp.max(match.astype(jnp.int32), axis=0, keepdims=True) > 0
    fidx = jnp.max(jnp.where(match, io_o, -1), axis=0, keepdims=True)
    idx_f = jnp.where(forced, fidx, bti)
    pos_row = jnp.logical_or(forced, bto >= _THRESHOLD)
    posf = pos_row.astype(jnp.float32)

    oh = (io_o == idx_f).astype(jnp.float32)
    tb = tg_ref[0]
    table = jnp.concatenate([tb[:, 0:4], tb[:, 5:15]], axis=1)
    matched = jax.lax.dot_general(
        table, oh, (((0,), (0,)), ((), ())),
        preferred_element_type=jnp.float32)

    pcxy = pt_ref[0:2, :]
    pwh = pt_ref[2:4, :]

    m12 = matched[0:2, :]
    m34 = matched[2:4, :]
    g_cxcy = ((m12 + m34) / 2.0 - pcxy) / (_VAR0 * pwh)
    g_wh = jnp.log((m34 - m12) / pwh) / _VAR1
    loc = loc_ref[0]
    sl = _smooth_l1(loc[0:2, :] - g_cxcy) + _smooth_l1(loc[2:4, :] - g_wh)
    loss_l = jnp.sum(sl * posf)

    ml = matched[4:14, :]
    pc_rep = jnp.concatenate([pcxy] * 5, axis=0)
    pwh_rep = jnp.concatenate([pwh] * 5, axis=0)
    g_lm = (ml - pc_rep) / (_VAR0 * pwh_rep)
    dlm = five_ref[0] - g_lm
    loss_coords = jnp.sum(dlm * dlm * posf)

    c = conf_ref[0]
    c0 = c[0:1, :]
    c1 = c[1:2, :]
    mx = jnp.maximum(c0, c1)
    lse = mx + jnp.log(jnp.exp(c0 - mx) + jnp.exp(c1 - mx))
    ce = lse - jnp.where(pos_row, c1, c0)
    pos_ce = jnp.sum(jnp.where(pos_row, ce, 0.0))
    negloss = jnp.where(pos_row, 0.0, lse - c0)
    rows_ref[0, 0] = negloss
    npos = jnp.sum(posf)

    lane = jax.lax.broadcasted_iota(jnp.int32, (1, 8), 1)
    vec = (jnp.where(lane == 0, loss_l, 0.0)
           + jnp.where(lane == 1, loss_coords, 0.0)
           + jnp.where(lane == 2, pos_ce, 0.0)
           + jnp.where(lane == 3, npos, 0.0))
    part_ref[0, 0] = vec


def _topk_kernel(rows_ref, part_ref, out_ref, *, P, B):
    v = rows_ref[...]
    bits = jax.lax.bitcast_convert_type(v, jnp.int32)
    npos = jnp.sum(part_ref[:, :, 0, 3], axis=1, keepdims=True)[..., None]
    k = jnp.minimum(_NEGPOS_RATIO * npos.astype(jnp.int32), P - 1)

    def body(_, carry):
        lo, hi = carry
        mid = lo + (hi - lo) // 2
        cnt = jnp.sum((bits >= mid).astype(jnp.int32), axis=(1, 2),
                      keepdims=True)
        good = cnt >= k
        return (jnp.where(good, mid, lo), jnp.where(good, hi, mid))

    z = jnp.zeros_like(k)
    lo, _ = jax.lax.fori_loop(0, 31, body, (z, z + 0x7F800000))
    tf = jax.lax.bitcast_convert_type(lo, jnp.float32)
    gt = bits > lo
    cnt_gt = jnp.sum(gt.astype(jnp.float32), axis=(1, 2), keepdims=True)
    sum_gt = jnp.sum(jnp.where(gt, v, 0.0), axis=(1, 2), keepdims=True)
    topk = sum_gt + (k.astype(jnp.float32) - cnt_gt) * tf
    topk = jnp.where(k >= 1, topk, 0.0)
    lane = jax.lax.broadcasted_iota(jnp.int32, (B, 1, 8), 2)
    out_ref[...] = jnp.where(lane == 0, topk, 0.0)


@jax.jit
def kernel(loc_data, conf_data, loc_five_data, priors, targets):
    B, P, _ = loc_data.shape
    O = targets.shape[1]
    TP = 16384 if P % 16384 == 0 else P
    T = P // TP

    priors_t = priors.T

    bto, bti, bpi = pl.pallas_call(
        functools.partial(_match_kernel, TP=TP, T=T, O=O),
        grid=(B, T),
        in_specs=[
            pl.BlockSpec((4, TP), lambda b, t: (0, t)),
            pl.BlockSpec((1, O, 15), lambda b, t: (b, 0, 0)),
        ],
        out_specs=[
            pl.BlockSpec((1, 1, 1, TP), lambda b, t: (b, t, 0, 0)),
            pl.BlockSpec((1, 1, 1, TP), lambda b, t: (b, t, 0, 0)),
            pl.BlockSpec((1, O, 1), lambda b, t: (b, 0, 0)),
        ],
        out_shape=[
            jax.ShapeDtypeStruct((B, T, 1, TP), jnp.float32),
            jax.ShapeDtypeStruct((B, T, 1, TP), jnp.int32),
            jax.ShapeDtypeStruct((B, O, 1), jnp.int32),
        ],
        scratch_shapes=[
            pltpu.VMEM((O, 1), jnp.float32),
            pltpu.VMEM((O, 1), jnp.int32),
        ],
    )(priors_t, targets)

    loc_t = loc_data.transpose(0, 2, 1)
    conf_t = conf_data.transpose(0, 2, 1)
    five_t = loc_five_data.transpose(0, 2, 1)

    rows, parts = pl.pallas_call(
        functools.partial(_loss_kernel, TP=TP, O=O),
        grid=(B, T),
        in_specs=[
            pl.BlockSpec((1, 4, TP), lambda b, t: (b, 0, t)),
            pl.BlockSpec((1, 2, TP), lambda b, t: (b, 0, t)),
            pl.BlockSpec((1, 10, TP), lambda b, t: (b, 0, t)),
            pl.BlockSpec((4, TP), lambda b, t: (0, t)),
            pl.BlockSpec((1, O, 15), lambda b, t: (b, 0, 0)),
            pl.BlockSpec((1, 1, 1, TP), lambda b, t: (b, t, 0, 0)),
            pl.BlockSpec((1, 1, 1, TP), lambda b, t: (b, t, 0, 0)),
            pl.BlockSpec((1, O, 1), lambda b, t: (b, 0, 0)),
        ],
        out_specs=[
            pl.BlockSpec((1, 1, 1, TP), lambda b, t: (b, t, 0, 0)),
            pl.BlockSpec((1, 1, 1, 8), lambda b, t: (b, t, 0, 0)),
        ],
        out_shape=[
            jax.ShapeDtypeStruct((B, T, 1, TP), jnp.float32),
            jax.ShapeDtypeStruct((B, T, 1, 8), jnp.float32),
        ],
    )(loc_t, conf_t, five_t, priors_t, targets, bto, bti, bpi)

    rows_r = rows.reshape(B, P // 128, 128)
    topk = pl.pallas_call(
        functools.partial(_topk_kernel, P=P, B=B),
        grid=(1,),
        in_specs=[
            pl.BlockSpec((B, P // 128, 128), lambda i: (0, 0, 0)),
            pl.BlockSpec((B, T, 1, 8), lambda i: (0, 0, 0, 0)),
        ],
        out_specs=pl.BlockSpec((B, 1, 8), lambda i: (0, 0, 0)),
        out_shape=jax.ShapeDtypeStruct((B, 1, 8), jnp.float32),
    )(rows_r, parts)

    loss_l = jnp.sum(parts[..., 0])
    loss_coords = jnp.sum(parts[..., 1])
    pos_ce = jnp.sum(parts[..., 2])
    np_tot = jnp.sum(parts[..., 3])
    topk_tot = jnp.sum(topk[:, 0, 0])
    n = jnp.maximum(np_tot, 1.0)
    return (loss_l / n, (pos_ce + topk_tot) / n, loss_coords / n)


# speedup vs baseline: 116.1534x; 1.0730x over previous
import functools

import jax
import jax.numpy as jnp
from jax.experimental import pallas as pl
from jax.experimental.pallas import tpu as pltpu

_THRESHOLD = 0.35
_NEGPOS_RATIO = 7
_VAR0 = 0.1
_VAR1 = 0.2


def _match_kernel(pt_ref, tg_ref, bto_ref, bti_ref, bpi_ref, sv_ref, si_ref,
                  *, TP, T, O):
    t = pl.program_id(1)

    @pl.when(t == 0)
    def _init():
        sv_ref[...] = jnp.full((O, 1), -1.0, jnp.float32)
        si_ref[...] = jnp.zeros((O, 1), jnp.int32)

    px = pt_ref[0:1, :]
    py = pt_ref[1:2, :]
    pw = pt_ref[2:3, :]
    ph = pt_ref[3:4, :]
    bx1 = px - pw / 2.0
    by1 = py - ph / 2.0
    bx2 = px + pw / 2.0
    by2 = py + ph / 2.0
    area_b = (bx2 - bx1) * (by2 - by1)

    tb = tg_ref[0]
    tx1 = tb[:, 0:1]
    ty1 = tb[:, 1:2]
    tx2 = tb[:, 2:3]
    ty2 = tb[:, 3:4]
    area_a = (tx2 - tx1) * (ty2 - ty1)

    iw = jnp.maximum(jnp.minimum(tx2, bx2) - jnp.maximum(tx1, bx1), 0.0)
    ih = jnp.maximum(jnp.minimum(ty2, by2) - jnp.maximum(ty1, by1), 0.0)
    inter = iw * ih
    iou = inter / (area_a + area_b - inter)

    io_o = jax.lax.broadcasted_iota(jnp.int32, (O, 1), 0)
    io_p = jax.lax.broadcasted_iota(jnp.int32, (1, TP), 1)

    pm = jnp.max(iou, axis=0, keepdims=True)
    pidx = jnp.min(jnp.where(iou == pm, io_o, O), axis=0, keepdims=True)
    bto_ref[0, 0] = pm
    bti_ref[0, 0] = pidx

    tv = jnp.max(iou, axis=1, keepdims=True)
    tl = jnp.min(jnp.where(iou == tv, io_p, TP), axis=1, keepdims=True)
    gidx = t * TP + tl
    sv = sv_ref[...]
    si = si_ref[...]
    upd = tv > sv
    new_sv = jnp.where(upd, tv, sv)
    new_si = jnp.where(upd, gidx, si)
    sv_ref[...] = new_sv
    si_ref[...] = new_si
    bpi_ref[0] = new_si


def _smooth_l1(d):
    ad = jnp.abs(d)
    return jnp.where(ad < 1.0, 0.5 * d * d, ad - 0.5)


def _loss_kernel(loc_ref, conf_ref, five_ref, pt_ref, tg_ref, bto_ref,
                 bti_ref, bpi_ref, rows_ref, part_ref, *, TP, O):
    t = pl.program_id(1)

    bto = bto_ref[0, 0]
    bti = bti_ref[0, 0]
    bpi = bpi_ref[0]

    io_o = jax.lax.broadcasted_iota(jnp.int32, (O, 1), 0)
    gp = t * TP + jax.lax.broadcasted_iota(jnp.int32, (1, TP), 1)

    match = bpi == gp
    fidx = jnp.max(jnp.where(match, io_o, -1), axis=0, keepdims=True)
    forced = fidx >= 0
    idx_f = jnp.where(forced, fidx, bti)
    pos_row = jnp.logical_or(forced, bto >= _THRESHOLD)
    posf = pos_row.astype(jnp.float32)

    oh = (io_o == idx_f).astype(jnp.float32)
    tb = tg_ref[0]
    table = jnp.concatenate([tb[:, 0:4], tb[:, 5:15]], axis=1)
    matched = jax.lax.dot_general(
        table, oh, (((0,), (0,)), ((), ())),
        preferred_element_type=jnp.float32)

    pcxy = pt_ref[0:2, :]
    pwh = pt_ref[2:4, :]

    inv_vwh = 1.0 / (_VAR0 * pwh)
    m12 = matched[0:2, :]
    m34 = matched[2:4, :]
    g_cxcy = ((m12 + m34) / 2.0 - pcxy) * inv_vwh
    g_wh = jnp.log((m34 - m12) / pwh) / _VAR1
    loc = loc_ref[0]
    sl = _smooth_l1(loc[0:2, :] - g_cxcy) + _smooth_l1(loc[2:4, :] - g_wh)
    loss_l = jnp.sum(sl * posf)

    ml = matched[4:14, :]
    pc_rep = jnp.concatenate([pcxy] * 5, axis=0)
    inv_rep = jnp.concatenate([inv_vwh] * 5, axis=0)
    g_lm = (ml - pc_rep) * inv_rep
    dlm = five_ref[0] - g_lm
    loss_coords = jnp.sum(dlm * dlm * posf)

    c = conf_ref[0]
    c0 = c[0:1, :]
    c1 = c[1:2, :]
    mx = jnp.maximum(c0, c1)
    lse = mx + jnp.log(jnp.exp(c0 - mx) + jnp.exp(c1 - mx))
    ce = lse - jnp.where(pos_row, c1, c0)
    pos_ce = jnp.sum(jnp.where(pos_row, ce, 0.0))
    negloss = jnp.where(pos_row, 0.0, lse - c0)
    rows_ref[0, 0] = negloss
    npos = jnp.sum(posf)

    lane = jax.lax.broadcasted_iota(jnp.int32, (1, 8), 1)
    vec = (jnp.where(lane == 0, loss_l, 0.0)
           + jnp.where(lane == 1, loss_coords, 0.0)
           + jnp.where(lane == 2, pos_ce, 0.0)
           + jnp.where(lane == 3, npos, 0.0))
    part_ref[0, 0] = vec


def _topk_kernel(rows_ref, part_ref, out_ref, *, P, B):
    v = rows_ref[...]
    bits = jax.lax.bitcast_convert_type(v, jnp.int32)
    npos = jnp.sum(part_ref[:, :, 0, 3], axis=1, keepdims=True)[..., None]
    k = jnp.minimum(_NEGPOS_RATIO * npos.astype(jnp.int32), P - 1)

    def body(_, carry):
        lo, hi = carry
        mid = lo + (hi - lo) // 2
        cnt = jnp.sum((bits >= mid).astype(jnp.int32), axis=(1, 2),
                      keepdims=True)
        good = cnt >= k
        return (jnp.where(good, mid, lo), jnp.where(good, hi, mid))

    z = jnp.zeros_like(k)
    lo, _ = jax.lax.fori_loop(0, 31, body, (z, z + 0x7F800000))
    tf = jax.lax.bitcast_convert_type(lo, jnp.float32)
    gt = bits > lo
    cnt_gt = jnp.sum(gt.astype(jnp.float32), axis=(1, 2), keepdims=True)
    sum_gt = jnp.sum(jnp.where(gt, v, 0.0), axis=(1, 2), keepdims=True)
    topk = sum_gt + (k.astype(jnp.float32) - cnt_gt) * tf
    topk = jnp.where(k >= 1, topk, 0.0)
    lane = jax.lax.broadcasted_iota(jnp.int32, (B, 1, 8), 2)
    out_ref[...] = jnp.where(lane == 0, topk, 0.0)


@jax.jit
def kernel(loc_data, conf_data, loc_five_data, priors, targets):
    B, P, _ = loc_data.shape
    O = targets.shape[1]
    TP = 16384 if P % 16384 == 0 else P
    T = P // TP

    priors_t = priors.T

    bto, bti, bpi = pl.pallas_call(
        functools.partial(_match_kernel, TP=TP, T=T, O=O),
        grid=(B, T),
        in_specs=[
            pl.BlockSpec((4, TP), lambda b, t: (0, t)),
            pl.BlockSpec((1, O, 15), lambda b, t: (b, 0, 0)),
        ],
        out_specs=[
            pl.BlockSpec((1, 1, 1, TP), lambda b, t: (b, t, 0, 0)),
            pl.BlockSpec((1, 1, 1, TP), lambda b, t: (b, t, 0, 0)),
            pl.BlockSpec((1, O, 1), lambda b, t: (b, 0, 0)),
        ],
        out_shape=[
            jax.ShapeDtypeStruct((B, T, 1, TP), jnp.float32),
            jax.ShapeDtypeStruct((B, T, 1, TP), jnp.int32),
            jax.ShapeDtypeStruct((B, O, 1), jnp.int32),
        ],
        scratch_shapes=[
            pltpu.VMEM((O, 1), jnp.float32),
            pltpu.VMEM((O, 1), jnp.int32),
        ],
    )(priors_t, targets)

    loc_t = loc_data.transpose(0, 2, 1)
    conf_t = conf_data.transpose(0, 2, 1)
    five_t = loc_five_data.transpose(0, 2, 1)

    rows, parts = pl.pallas_call(
        functools.partial(_loss_kernel, TP=TP, O=O),
        grid=(B, T),
        in_specs=[
            pl.BlockSpec((1, 4, TP), lambda b, t: (b, 0, t)),
            pl.BlockSpec((1, 2, TP), lambda b, t: (b, 0, t)),
            pl.BlockSpec((1, 10, TP), lambda b, t: (b, 0, t)),
            pl.BlockSpec((4, TP), lambda b, t: (0, t)),
            pl.BlockSpec((1, O, 15), lambda b, t: (b, 0, 0)),
            pl.BlockSpec((1, 1, 1, TP), lambda b, t: (b, t, 0, 0)),
            pl.BlockSpec((1, 1, 1, TP), lambda b, t: (b, t, 0, 0)),
            pl.BlockSpec((1, O, 1), lambda b, t: (b, 0, 0)),
        ],
        out_specs=[
            pl.BlockSpec((1, 1, 1, TP), lambda b, t: (b, t, 0, 0)),
            pl.BlockSpec((1, 1, 1, 8), lambda b, t: (b, t, 0, 0)),
        ],
        out_shape=[
            jax.ShapeDtypeStruct((B, T, 1, TP), jnp.float32),
            jax.ShapeDtypeStruct((B, T, 1, 8), jnp.float32),
        ],
    )(loc_t, conf_t, five_t, priors_t, targets, bto, bti, bpi)

    rows_r = rows.reshape(B, P // 128, 128)
    topk = pl.pallas_call(
        functools.partial(_topk_kernel, P=P, B=B),
        grid=(1,),
        in_specs=[
            pl.BlockSpec((B, P // 128, 128), lambda i: (0, 0, 0)),
            pl.BlockSpec((B, T, 1, 8), lambda i: (0, 0, 0, 0)),
        ],
        out_specs=pl.BlockSpec((B, 1, 8), lambda i: (0, 0, 0)),
        out_shape=jax.ShapeDtypeStruct((B, 1, 8), jnp.float32),
    )(rows_r, parts)

    loss_l = jnp.sum(parts[..., 0])
    loss_coords = jnp.sum(parts[..., 1])
    pos_ce = jnp.sum(parts[..., 2])
    np_tot = jnp.sum(parts[..., 3])
    topk_tot = jnp.sum(topk[:, 0, 0])
    n = jnp.maximum(np_tot, 1.0)
    return (loss_l / n, (pos_ce + topk_tot) / n, loss_coords / n)
